# baseline (speedup 1.0000x reference)
import functools

import jax
import jax.numpy as jnp
from jax import lax
from jax.experimental import pallas as pl
from jax.experimental.pallas import tpu as pltpu
from jax.experimental.pallas import tpu_sc as plsc

_MAXLEN = 2048
_NC, _NS = 2, 16
_NW = _NC * _NS
_CHUNK = 128


def _main_body(x_ref, pt_ref, et_ref, dur_ref, pb_ref, eb_ref, ptab_ref,
               etab_ref, wa_ref, wb_ref, vecs_ref, lb_ref,
               x3_ref, g_ref, m_ref, ld_ref, pp_ref, ep_ref):
    b = pl.program_id(0)
    xb = x_ref[0]
    src, d = xb.shape
    pv = pt_ref[0]
    ev = et_ref[0]
    durc = dur_ref[0].astype(jnp.float32)
    pb = pb_ref[...]
    eb = eb_ref[...]
    nb = pb.shape[1]

    pcnt = jnp.sum((pb <= pv).astype(jnp.int32), axis=1, keepdims=True)
    ecnt = jnp.sum((eb <= ev).astype(jnp.int32), axis=1, keepdims=True)
    lane = lax.broadcasted_iota(jnp.int32, (src, nb), 1)
    ph = (lane == pcnt).astype(jnp.float32)
    eh = (lane == ecnt).astype(jnp.float32)
    x2 = xb + jnp.dot(ph, ptab_ref[...], preferred_element_type=jnp.float32)
    x3 = x2 + jnp.dot(eh, etab_ref[...], preferred_element_type=jnp.float32)
    x3_ref[...] = x3

    ii = lax.broadcasted_iota(jnp.int32, (src, src), 0)
    jj = lax.broadcasted_iota(jnp.int32, (src, src), 1)
    tlow = (jj <= ii).astype(jnp.float32)
    cum = jnp.dot(tlow, durc, preferred_element_type=jnp.float32)

    maxl = g_ref.shape[2]
    frow = lax.broadcasted_iota(jnp.int32, (1, maxl), 1).astype(jnp.float32)
    cmp = (cum <= frow).astype(jnp.int32)
    idx = jnp.sum(cmp, axis=0, keepdims=True)
    msk = (idx >= src).astype(jnp.int32)
    g = jnp.where(msk == 1, 2 * src, (b % 2) * src + idx)
    g_ref[0] = g
    m_ref[0] = msk

    ld_ref[0] = _one_predictor(xb, wa_ref, wb_ref, vecs_ref, lb_ref, 0)
    pp_ref[0] = _one_predictor(xb, wa_ref, wb_ref, vecs_ref, lb_ref, 1)
    ep_ref[0] = _one_predictor(x2, wa_ref, wb_ref, vecs_ref, lb_ref, 2)


def _run_main(x, pt3, et3, dur3, pbrow, ebrow, ptab, etab, wa, wb, vecs, lbs):
    b, src, d = x.shape
    nb = pbrow.shape[1]
    const2 = lambda i: (0, 0)
    c4 = lambda i: (0, 0, 0, 0)
    out = pl.pallas_call(
        _main_body,
        grid=(b,),
        in_specs=[
            pl.BlockSpec((1, src, d), lambda i: (i, 0, 0)),
            pl.BlockSpec((1, src, 1), lambda i: (i, 0, 0)),
            pl.BlockSpec((1, src, 1), lambda i: (i, 0, 0)),
            pl.BlockSpec((1, src, 1), lambda i: (i, 0, 0)),
            pl.BlockSpec((1, nb), const2),
            pl.BlockSpec((1, nb), const2),
            pl.BlockSpec((nb, d), const2),
            pl.BlockSpec((nb, d), const2),
            pl.BlockSpec((3, 3, d, d), c4),
            pl.BlockSpec((3, 3, d, d), c4),
            pl.BlockSpec((3, 7, 1, d), c4),
            pl.BlockSpec((3, 1, 1), lambda i: (0, 0, 0)),
        ],
        out_specs=[
            pl.BlockSpec((src, d), lambda i: (i, 0)),
            pl.BlockSpec((1, 1, _MAXLEN), lambda i: (i, 0, 0)),
            pl.BlockSpec((1, 1, _MAXLEN), lambda i: (i, 0, 0)),
            pl.BlockSpec((1, src, 1), lambda i: (i, 0, 0)),
            pl.BlockSpec((1, src, 1), lambda i: (i, 0, 0)),
            pl.BlockSpec((1, src, 1), lambda i: (i, 0, 0)),
        ],
        out_shape=[
            jax.ShapeDtypeStruct((b * src, d), jnp.float32),
            jax.ShapeDtypeStruct((b, 1, _MAXLEN), jnp.int32),
            jax.ShapeDtypeStruct((b, 1, _MAXLEN), jnp.int32),
            jax.ShapeDtypeStruct((b, src, 1), jnp.float32),
            jax.ShapeDtypeStruct((b, src, 1), jnp.float32),
            jax.ShapeDtypeStruct((b, src, 1), jnp.float32),
        ],
    )(x, pt3, et3, dur3, pbrow, ebrow, ptab, etab, wa, wb, vecs, lbs)
    return out


def _shift_down(a):
    return jnp.concatenate([jnp.zeros_like(a[:1]), a[:-1]], axis=0)


def _shift_up(a):
    return jnp.concatenate([a[1:], jnp.zeros_like(a[:1])], axis=0)


def _conv3(h, w_ref, p):
    a0 = jnp.dot(h, w_ref[p, 0], preferred_element_type=jnp.float32)
    a1 = jnp.dot(h, w_ref[p, 1], preferred_element_type=jnp.float32)
    a2 = jnp.dot(h, w_ref[p, 2], preferred_element_type=jnp.float32)
    return _shift_down(a0) + a1 + _shift_up(a2)


def _ln(h, g, be):
    mu = jnp.mean(h, axis=1, keepdims=True)
    dd = h - mu
    var = jnp.mean(dd * dd, axis=1, keepdims=True)
    return g * dd * lax.rsqrt(var + 1e-5) + be


def _one_predictor(h0, wa_ref, wb_ref, vecs_ref, lb_ref, p):
    y1 = _conv3(h0, wa_ref, p) + vecs_ref[p, 0]
    h1 = _ln(jnp.maximum(y1, 0.0), vecs_ref[p, 1], vecs_ref[p, 2])
    y2 = _conv3(h1, wb_ref, p) + vecs_ref[p, 3]
    h2 = _ln(jnp.maximum(y2, 0.0), vecs_ref[p, 4], vecs_ref[p, 5])
    pred = jnp.sum(h2 * vecs_ref[p, 6], axis=1, keepdims=True) + lb_ref[p]
    return pred


@functools.lru_cache(maxsize=None)
def _make_sc_gather(n_rows, d, src, n_sent):
    nphase = 2
    n_pair = 2 * src
    stage = n_pair // _NS
    per_tile = n_rows // (2 * nphase * _NS)
    nchunks = per_tile // _CHUNK
    mesh = plsc.VectorSubcoreMesh(core_axis_name="c", subcore_axis_name="s")

    @functools.partial(
        pl.kernel,
        mesh=mesh,
        out_type=jax.ShapeDtypeStruct((n_rows, d), jnp.float32),
        scratch_types=[
            pltpu.VMEM_SHARED((n_pair + n_sent, d), jnp.float32),
            pltpu.VMEM_SHARED((_NS, nphase * nchunks * _CHUNK), jnp.int32),
            pltpu.SMEM((nphase * nchunks * _CHUNK,), jnp.int32),
            pltpu.VMEM((_CHUNK, d), jnp.float32),
            pltpu.VMEM((_CHUNK, d), jnp.float32),
            pltpu.SemaphoreType.DMA,
            pltpu.SemaphoreType.DMA,
        ],
    )
    def sc_gather(table_hbm, zrows_hbm, ids_hbm, out_hbm, tab_sh, ids_sh,
                  ids_sm, b0, b1, s0, s1):
        sid = lax.axis_index("s")
        c = lax.axis_index("c")
        pltpu.sync_copy(ids_hbm.at[c, sid], ids_sh.at[sid])
        pltpu.sync_copy(ids_sh.at[sid], ids_sm)

        @pl.when(sid == 0)
        def _():
            pltpu.sync_copy(zrows_hbm, tab_sh.at[pl.ds(n_pair, n_sent)])

        bufs = (b0, b1)
        sems = (s0, s1)
        for p in range(nphase):
            pltpu.sync_copy(
                table_hbm.at[pl.ds((2 * c + p) * n_pair + sid * stage,
                                   stage)],
                tab_sh.at[pl.ds(sid * stage, stage)])
            plsc.subcore_barrier()
            base = (c * nphase + p) * (_NS * per_tile) + sid * per_tile
            for k in range(nchunks):
                row0 = p * nchunks + k

                def _fire(j, _, _k=k, _row0=row0):
                    rid = ids_sm[_row0 * _CHUNK + j]
                    pltpu.async_copy(tab_sh.at[pl.ds(rid, 1)],
                                     bufs[_k].at[pl.ds(j, 1)], sems[_k])
                    return _

                lax.fori_loop(0, _CHUNK, _fire, 0)
            for k in range(nchunks):
                pltpu.make_async_copy(table_hbm.at[pl.ds(0, _CHUNK)],
                                      bufs[k], sems[k]).wait()
                pltpu.sync_copy(bufs[k],
                                out_hbm.at[pl.ds(base + k * _CHUNK, _CHUNK)])
            plsc.subcore_barrier()

    return sc_gather


def kernel(x, src_mask, duration_target, pitch_target, energy_target,
           max_len, params, pitch_bins, energy_bins):
    b, src, d = x.shape
    ptab = params["pitch_emb"]
    etab = params["energy_emb"]
    nbins = ptab.shape[0]

    pt3 = pitch_target[:, :, None]
    et3 = energy_target[:, :, None]
    dur3 = duration_target[:, :, None]
    pad = jnp.full((nbins - pitch_bins.shape[0],), jnp.inf, jnp.float32)
    pbrow = jnp.concatenate([pitch_bins, pad])[None, :]
    ebrow = jnp.concatenate([energy_bins, pad])[None, :]

    def _vp_pack(p):
        wa = jnp.transpose(p["w1"], (2, 1, 0))
        wb = jnp.transpose(p["w2"], (2, 1, 0))
        vecs = jnp.stack([p["b1"], p["g1"], p["be1"], p["b2"], p["g2"],
                          p["be2"], p["lw"]])[:, None, :]
        return wa, wb, vecs, p["lb"].reshape(1, 1)

    packs = [_vp_pack(params[k]) for k in ("duration", "pitch", "energy")]
    wa = jnp.stack([pk[0] for pk in packs])
    wb = jnp.stack([pk[1] for pk in packs])
    vecs = jnp.stack([pk[2] for pk in packs])
    lbs = jnp.stack([pk[3] for pk in packs])

    x3flat, g, mask_i, ld, pp, ep = _run_main(
        x, pt3, et3, dur3, pbrow, ebrow, ptab, etab, wa, wb, vecs, lbs)

    n_sent = 8
    zrows = jnp.zeros((n_sent, d), x.dtype)
    n_rows = b * _MAXLEN
    g_tiled = (g.reshape(2, 2, _NS, 2, _CHUNK)
                .transpose(0, 2, 1, 3, 4)
                .reshape(2, _NS, 4 * _CHUNK))
    out_flat = _make_sc_gather(n_rows, d, src, n_sent)(x3flat, zrows, g_tiled)
    out = out_flat.reshape(b, _MAXLEN, d)
    log_dur = jnp.where(src_mask, 0.0, ld[:, :, 0])
    pitch_pred = jnp.where(src_mask, 0.0, pp[:, :, 0])
    energy_pred = jnp.where(src_mask, 0.0, ep[:, :, 0])

    total = jnp.sum(duration_target, axis=1)
    mel_len = jnp.minimum(total, max_len)
    mel_mask = mask_i.reshape(b, _MAXLEN).astype(bool)

    return out, pitch_pred, energy_pred, log_dur, mel_len, mel_mask


# speedup vs baseline: 1.1220x; 1.1220x over previous
import functools

import jax
import jax.numpy as jnp
from jax import lax
from jax.experimental import pallas as pl
from jax.experimental.pallas import tpu as pltpu
from jax.experimental.pallas import tpu_sc as plsc

_MAXLEN = 2048
_NC, _NS = 2, 16
_NW = _NC * _NS
_CHUNK = 128


def _idx_body(x_ref, pt_ref, et_ref, dur_ref, pb_ref, eb_ref, ptab_ref,
              etab_ref, x2_ref, x3_ref, g_ref, m_ref):
    b = pl.program_id(0)
    xb = x_ref[0]
    src, d = xb.shape
    pv = pt_ref[0]
    ev = et_ref[0]
    durc = dur_ref[0].astype(jnp.float32)
    pb = pb_ref[...]
    eb = eb_ref[...]
    nb = pb.shape[1]

    pcnt = jnp.sum((pb <= pv).astype(jnp.int32), axis=1, keepdims=True)
    ecnt = jnp.sum((eb <= ev).astype(jnp.int32), axis=1, keepdims=True)
    lane = lax.broadcasted_iota(jnp.int32, (src, nb), 1)
    ph = (lane == pcnt).astype(jnp.float32)
    eh = (lane == ecnt).astype(jnp.float32)
    x2 = xb + jnp.dot(ph, ptab_ref[...], preferred_element_type=jnp.float32)
    x3 = x2 + jnp.dot(eh, etab_ref[...], preferred_element_type=jnp.float32)
    x2_ref[0] = x2
    x3_ref[...] = x3

    ii = lax.broadcasted_iota(jnp.int32, (src, src), 0)
    jj = lax.broadcasted_iota(jnp.int32, (src, src), 1)
    tlow = (jj <= ii).astype(jnp.float32)
    cum = jnp.dot(tlow, durc, preferred_element_type=jnp.float32)

    maxl = g_ref.shape[2]
    frow = lax.broadcasted_iota(jnp.int32, (1, maxl), 1).astype(jnp.float32)
    cmp = (cum <= frow).astype(jnp.int32)
    idx = jnp.sum(cmp, axis=0, keepdims=True)
    msk = (idx >= src).astype(jnp.int32)
    g = jnp.where(msk == 1, 2 * src, (b % 2) * src + idx)
    g_ref[0] = g
    m_ref[0] = msk


def _run_idx(x, pt3, et3, dur3, pbrow, ebrow, ptab, etab):
    b, src, d = x.shape
    nb = pbrow.shape[1]
    const2 = lambda i: (0, 0)
    out = pl.pallas_call(
        _idx_body,
        grid=(b,),
        in_specs=[
            pl.BlockSpec((1, src, d), lambda i: (i, 0, 0)),
            pl.BlockSpec((1, src, 1), lambda i: (i, 0, 0)),
            pl.BlockSpec((1, src, 1), lambda i: (i, 0, 0)),
            pl.BlockSpec((1, src, 1), lambda i: (i, 0, 0)),
            pl.BlockSpec((1, nb), const2),
            pl.BlockSpec((1, nb), const2),
            pl.BlockSpec((nb, d), const2),
            pl.BlockSpec((nb, d), const2),
        ],
        out_specs=[
            pl.BlockSpec((1, src, d), lambda i: (i, 0, 0)),
            pl.BlockSpec((src, d), lambda i: (i, 0)),
            pl.BlockSpec((1, 1, _MAXLEN), lambda i: (i, 0, 0)),
            pl.BlockSpec((1, 1, _MAXLEN), lambda i: (i, 0, 0)),
        ],
        out_shape=[
            jax.ShapeDtypeStruct((b, src, d), jnp.float32),
            jax.ShapeDtypeStruct((b * src, d), jnp.float32),
            jax.ShapeDtypeStruct((b, 1, _MAXLEN), jnp.int32),
            jax.ShapeDtypeStruct((b, 1, _MAXLEN), jnp.int32),
        ],
    )(x, pt3, et3, dur3, pbrow, ebrow, ptab, etab)
    return out


def _pred_body(x_ref, x2_ref, wa_ref, wb_ref, vecs_ref, lb_ref,
               ld_ref, pp_ref, ep_ref):
    xb = x_ref[0]
    x2b = x2_ref[0]
    ld_ref[0] = _one_predictor(xb, wa_ref, wb_ref, vecs_ref, lb_ref, 0)
    pp_ref[0] = _one_predictor(xb, wa_ref, wb_ref, vecs_ref, lb_ref, 1)
    ep_ref[0] = _one_predictor(x2b, wa_ref, wb_ref, vecs_ref, lb_ref, 2)


def _run_preds(x, x2, wa, wb, vecs, lbs):
    b, src, d = x.shape
    c4 = lambda i: (0, 0, 0, 0)
    out = pl.pallas_call(
        _pred_body,
        grid=(b,),
        in_specs=[
            pl.BlockSpec((1, src, d), lambda i: (i, 0, 0)),
            pl.BlockSpec((1, src, d), lambda i: (i, 0, 0)),
            pl.BlockSpec((3, 3, d, d), c4),
            pl.BlockSpec((3, 3, d, d), c4),
            pl.BlockSpec((3, 7, 1, d), c4),
            pl.BlockSpec((3, 1, 1), lambda i: (0, 0, 0)),
        ],
        out_specs=[
            pl.BlockSpec((1, src, 1), lambda i: (i, 0, 0)),
            pl.BlockSpec((1, src, 1), lambda i: (i, 0, 0)),
            pl.BlockSpec((1, src, 1), lambda i: (i, 0, 0)),
        ],
        out_shape=[
            jax.ShapeDtypeStruct((b, src, 1), jnp.float32),
            jax.ShapeDtypeStruct((b, src, 1), jnp.float32),
            jax.ShapeDtypeStruct((b, src, 1), jnp.float32),
        ],
    )(x, x2, wa, wb, vecs, lbs)
    return out


def _shift_down(a):
    return jnp.concatenate([jnp.zeros_like(a[:1]), a[:-1]], axis=0)


def _shift_up(a):
    return jnp.concatenate([a[1:], jnp.zeros_like(a[:1])], axis=0)


def _conv3(h, w_ref, p):
    a0 = jnp.dot(h, w_ref[p, 0], preferred_element_type=jnp.float32)
    a1 = jnp.dot(h, w_ref[p, 1], preferred_element_type=jnp.float32)
    a2 = jnp.dot(h, w_ref[p, 2], preferred_element_type=jnp.float32)
    return _shift_down(a0) + a1 + _shift_up(a2)


def _ln(h, g, be):
    mu = jnp.mean(h, axis=1, keepdims=True)
    dd = h - mu
    var = jnp.mean(dd * dd, axis=1, keepdims=True)
    return g * dd * lax.rsqrt(var + 1e-5) + be


def _one_predictor(h0, wa_ref, wb_ref, vecs_ref, lb_ref, p):
    y1 = _conv3(h0, wa_ref, p) + vecs_ref[p, 0]
    h1 = _ln(jnp.maximum(y1, 0.0), vecs_ref[p, 1], vecs_ref[p, 2])
    y2 = _conv3(h1, wb_ref, p) + vecs_ref[p, 3]
    h2 = _ln(jnp.maximum(y2, 0.0), vecs_ref[p, 4], vecs_ref[p, 5])
    pred = jnp.sum(h2 * vecs_ref[p, 6], axis=1, keepdims=True) + lb_ref[p]
    return pred


@functools.lru_cache(maxsize=None)
def _make_sc_gather(n_rows, d, src, n_sent):
    nphase = 2
    n_pair = 2 * src
    stage = n_pair // _NS
    per_tile = n_rows // (2 * nphase * _NS)
    nchunks = per_tile // _CHUNK
    mesh = plsc.VectorSubcoreMesh(core_axis_name="c", subcore_axis_name="s")

    @functools.partial(
        pl.kernel,
        mesh=mesh,
        out_type=jax.ShapeDtypeStruct((n_rows, d), jnp.float32),
        scratch_types=[
            pltpu.VMEM_SHARED((n_pair + n_sent, d), jnp.float32),
            pltpu.VMEM_SHARED((_NS, nphase * nchunks * _CHUNK), jnp.int32),
            pltpu.SMEM((nphase * nchunks * _CHUNK,), jnp.int32),
            pltpu.VMEM((_CHUNK, d), jnp.float32),
            pltpu.VMEM((_CHUNK, d), jnp.float32),
            pltpu.SemaphoreType.DMA,
            pltpu.SemaphoreType.DMA,
        ],
    )
    def sc_gather(table_hbm, zrows_hbm, ids_hbm, out_hbm, tab_sh, ids_sh,
                  ids_sm, b0, b1, s0, s1):
        sid = lax.axis_index("s")
        c = lax.axis_index("c")
        pltpu.sync_copy(ids_hbm.at[c, sid], ids_sh.at[sid])
        pltpu.sync_copy(ids_sh.at[sid], ids_sm)

        @pl.when(sid == 0)
        def _():
            pltpu.sync_copy(zrows_hbm, tab_sh.at[pl.ds(n_pair, n_sent)])

        bufs = (b0, b1)
        sems = (s0, s1)
        for p in range(nphase):
            pltpu.sync_copy(
                table_hbm.at[pl.ds((2 * c + p) * n_pair + sid * stage,
                                   stage)],
                tab_sh.at[pl.ds(sid * stage, stage)])
            plsc.subcore_barrier()
            base = (c * nphase + p) * (_NS * per_tile) + sid * per_tile
            for k in range(nchunks):
                row0 = p * nchunks + k

                def _fire(j, _, _k=k, _row0=row0):
                    rid = ids_sm[_row0 * _CHUNK + j]
                    pltpu.async_copy(tab_sh.at[pl.ds(rid, 1)],
                                     bufs[_k].at[pl.ds(j, 1)], sems[_k])
                    return _

                lax.fori_loop(0, _CHUNK, _fire, 0)
            for k in range(nchunks):
                pltpu.make_async_copy(table_hbm.at[pl.ds(0, _CHUNK)],
                                      bufs[k], sems[k]).wait()
                pltpu.sync_copy(bufs[k],
                                out_hbm.at[pl.ds(base + k * _CHUNK, _CHUNK)])
            plsc.subcore_barrier()

    return sc_gather


def kernel(x, src_mask, duration_target, pitch_target, energy_target,
           max_len, params, pitch_bins, energy_bins):
    b, src, d = x.shape
    ptab = params["pitch_emb"]
    etab = params["energy_emb"]
    nbins = ptab.shape[0]

    pt3 = pitch_target[:, :, None]
    et3 = energy_target[:, :, None]
    dur3 = duration_target[:, :, None]
    pad = jnp.full((nbins - pitch_bins.shape[0],), jnp.inf, jnp.float32)
    pbrow = jnp.concatenate([pitch_bins, pad])[None, :]
    ebrow = jnp.concatenate([energy_bins, pad])[None, :]

    def _vp_pack(p):
        wa = jnp.transpose(p["w1"], (2, 1, 0))
        wb = jnp.transpose(p["w2"], (2, 1, 0))
        vecs = jnp.stack([p["b1"], p["g1"], p["be1"], p["b2"], p["g2"],
                          p["be2"], p["lw"]])[:, None, :]
        return wa, wb, vecs, p["lb"].reshape(1, 1)

    packs = [_vp_pack(params[k]) for k in ("duration", "pitch", "energy")]
    wa = jnp.stack([pk[0] for pk in packs])
    wb = jnp.stack([pk[1] for pk in packs])
    vecs = jnp.stack([pk[2] for pk in packs])
    lbs = jnp.stack([pk[3] for pk in packs])

    x2, x3flat, g, mask_i = _run_idx(x, pt3, et3, dur3, pbrow, ebrow,
                                     ptab, etab)

    n_sent = 8
    zrows = jnp.zeros((n_sent, d), x.dtype)
    n_rows = b * _MAXLEN
    g_tiled = (g.reshape(2, 2, _NS, 2, _CHUNK)
                .transpose(0, 2, 1, 3, 4)
                .reshape(2, _NS, 4 * _CHUNK))
    out_flat = _make_sc_gather(n_rows, d, src, n_sent)(x3flat, zrows, g_tiled)
    out = out_flat.reshape(b, _MAXLEN, d)

    ld, pp, ep = _run_preds(x, x2, wa, wb, vecs, lbs)
    log_dur = jnp.where(src_mask, 0.0, ld[:, :, 0])
    pitch_pred = jnp.where(src_mask, 0.0, pp[:, :, 0])
    energy_pred = jnp.where(src_mask, 0.0, ep[:, :, 0])

    total = jnp.sum(duration_target, axis=1)
    mel_len = jnp.minimum(total, max_len)
    mel_mask = mask_i.reshape(b, _MAXLEN).astype(bool)

    return out, pitch_pred, energy_pred, log_dur, mel_len, mel_mask


# speedup vs baseline: 1.1250x; 1.0027x over previous
import functools

import jax
import jax.numpy as jnp
from jax import lax
from jax.experimental import pallas as pl
from jax.experimental.pallas import tpu as pltpu
from jax.experimental.pallas import tpu_sc as plsc

_MAXLEN = 2048
_NC, _NS = 2, 16
_NW = _NC * _NS
_CHUNK = 128


def _idx_body(x_ref, pt_ref, et_ref, dur_ref, pb_ref, eb_ref, ptab_ref,
              etab_ref, x2_ref, x3_ref, g_ref, m_ref):
    b = pl.program_id(0)
    xb = x_ref[0]
    src, d = xb.shape
    pv = pt_ref[0]
    ev = et_ref[0]
    durc = dur_ref[0].astype(jnp.float32)
    pb = pb_ref[...]
    eb = eb_ref[...]
    nb = pb.shape[1]

    pcnt = jnp.sum((pb <= pv).astype(jnp.int32), axis=1, keepdims=True)
    ecnt = jnp.sum((eb <= ev).astype(jnp.int32), axis=1, keepdims=True)
    lane = lax.broadcasted_iota(jnp.int32, (src, nb), 1)
    ph = (lane == pcnt).astype(jnp.float32)
    eh = (lane == ecnt).astype(jnp.float32)
    x2 = xb + jnp.dot(ph, ptab_ref[...], preferred_element_type=jnp.float32)
    x3 = x2 + jnp.dot(eh, etab_ref[...], preferred_element_type=jnp.float32)
    x2_ref[0] = x2
    x3_ref[...] = x3

    ii = lax.broadcasted_iota(jnp.int32, (src, src), 0)
    jj = lax.broadcasted_iota(jnp.int32, (src, src), 1)
    tlow = (jj <= ii).astype(jnp.float32)
    cum = jnp.dot(tlow, durc, preferred_element_type=jnp.float32)

    maxl = g_ref.shape[2]
    frow = lax.broadcasted_iota(jnp.int32, (1, maxl), 1).astype(jnp.float32)
    cmp = (cum <= frow).astype(jnp.int32)
    idx = jnp.sum(cmp, axis=0, keepdims=True)
    msk = (idx >= src).astype(jnp.int32)
    g = jnp.where(msk == 1, 2 * src, (b % 2) * src + idx)
    g_ref[0] = g
    m_ref[0] = msk


def _run_idx(x, pt3, et3, dur3, pbrow, ebrow, ptab, etab):
    b, src, d = x.shape
    nb = pbrow.shape[1]
    const2 = lambda i: (0, 0)
    out = pl.pallas_call(
        _idx_body,
        grid=(b,),
        in_specs=[
            pl.BlockSpec((1, src, d), lambda i: (i, 0, 0)),
            pl.BlockSpec((1, src, 1), lambda i: (i, 0, 0)),
            pl.BlockSpec((1, src, 1), lambda i: (i, 0, 0)),
            pl.BlockSpec((1, src, 1), lambda i: (i, 0, 0)),
            pl.BlockSpec((1, nb), const2),
            pl.BlockSpec((1, nb), const2),
            pl.BlockSpec((nb, d), const2),
            pl.BlockSpec((nb, d), const2),
        ],
        out_specs=[
            pl.BlockSpec((1, src, d), lambda i: (i, 0, 0)),
            pl.BlockSpec((src, d), lambda i: (i, 0)),
            pl.BlockSpec((1, 1, _MAXLEN), lambda i: (i, 0, 0)),
            pl.BlockSpec((1, 1, _MAXLEN), lambda i: (i, 0, 0)),
        ],
        out_shape=[
            jax.ShapeDtypeStruct((b, src, d), jnp.float32),
            jax.ShapeDtypeStruct((b * src, d), jnp.float32),
            jax.ShapeDtypeStruct((b, 1, _MAXLEN), jnp.int32),
            jax.ShapeDtypeStruct((b, 1, _MAXLEN), jnp.int32),
        ],
    )(x, pt3, et3, dur3, pbrow, ebrow, ptab, etab)
    return out


def _pred_body(x_ref, x2_ref, wa_ref, wb_ref, vecs_ref, lb_ref,
               ld_ref, pp_ref, ep_ref):
    xb = x_ref[0]
    x2b = x2_ref[0]
    ld_ref[0] = _one_predictor(xb, wa_ref, wb_ref, vecs_ref, lb_ref, 0)
    pp_ref[0] = _one_predictor(xb, wa_ref, wb_ref, vecs_ref, lb_ref, 1)
    ep_ref[0] = _one_predictor(x2b, wa_ref, wb_ref, vecs_ref, lb_ref, 2)


def _run_preds(x, x2, wa, wb, vecs, lbs):
    b, src, d = x.shape
    c4 = lambda i: (0, 0, 0, 0)
    out = pl.pallas_call(
        _pred_body,
        grid=(b,),
        in_specs=[
            pl.BlockSpec((1, src, d), lambda i: (i, 0, 0)),
            pl.BlockSpec((1, src, d), lambda i: (i, 0, 0)),
            pl.BlockSpec((3, 3, d, d), c4),
            pl.BlockSpec((3, 3, d, d), c4),
            pl.BlockSpec((3, 7, 1, d), c4),
            pl.BlockSpec((3, 1, 1), lambda i: (0, 0, 0)),
        ],
        out_specs=[
            pl.BlockSpec((1, src, 1), lambda i: (i, 0, 0)),
            pl.BlockSpec((1, src, 1), lambda i: (i, 0, 0)),
            pl.BlockSpec((1, src, 1), lambda i: (i, 0, 0)),
        ],
        out_shape=[
            jax.ShapeDtypeStruct((b, src, 1), jnp.float32),
            jax.ShapeDtypeStruct((b, src, 1), jnp.float32),
            jax.ShapeDtypeStruct((b, src, 1), jnp.float32),
        ],
    )(x, x2, wa, wb, vecs, lbs)
    return out


def _shift_down(a):
    return jnp.concatenate([jnp.zeros_like(a[:1]), a[:-1]], axis=0)


def _shift_up(a):
    return jnp.concatenate([a[1:], jnp.zeros_like(a[:1])], axis=0)


def _conv3(h, w_ref, p):
    a0 = jnp.dot(h, w_ref[p, 0], preferred_element_type=jnp.float32)
    a1 = jnp.dot(h, w_ref[p, 1], preferred_element_type=jnp.float32)
    a2 = jnp.dot(h, w_ref[p, 2], preferred_element_type=jnp.float32)
    return _shift_down(a0) + a1 + _shift_up(a2)


def _ln(h, g, be):
    mu = jnp.mean(h, axis=1, keepdims=True)
    dd = h - mu
    var = jnp.mean(dd * dd, axis=1, keepdims=True)
    return g * dd * lax.rsqrt(var + 1e-5) + be


def _one_predictor(h0, wa_ref, wb_ref, vecs_ref, lb_ref, p):
    y1 = _conv3(h0, wa_ref, p) + vecs_ref[p, 0]
    h1 = _ln(jnp.maximum(y1, 0.0), vecs_ref[p, 1], vecs_ref[p, 2])
    y2 = _conv3(h1, wb_ref, p) + vecs_ref[p, 3]
    h2 = _ln(jnp.maximum(y2, 0.0), vecs_ref[p, 4], vecs_ref[p, 5])
    pred = jnp.sum(h2 * vecs_ref[p, 6], axis=1, keepdims=True) + lb_ref[p]
    return pred


@functools.lru_cache(maxsize=None)
def _make_sc_gather(n_rows, d, src, n_sent):
    nphase = 2
    n_pair = 2 * src
    stage = n_pair // _NS
    per_tile = n_rows // (2 * nphase * _NS)
    nchunks = per_tile // _CHUNK
    mesh = plsc.VectorSubcoreMesh(core_axis_name="c", subcore_axis_name="s")

    @functools.partial(
        pl.kernel,
        mesh=mesh,
        out_type=jax.ShapeDtypeStruct((n_rows, d), jnp.float32),
        scratch_types=[
            pltpu.VMEM_SHARED((n_pair + n_sent, d), jnp.float32),
            pltpu.VMEM_SHARED((_NS, nphase * nchunks * _CHUNK), jnp.int32),
            pltpu.SMEM((nphase * nchunks * _CHUNK,), jnp.int32),
            pltpu.VMEM((_CHUNK, d), jnp.float32),
            pltpu.VMEM((_CHUNK, d), jnp.float32),
            pltpu.SemaphoreType.DMA,
            pltpu.SemaphoreType.DMA,
        ],
    )
    def sc_gather(table_hbm, zrows_hbm, ids_hbm, out_hbm, tab_sh, ids_sh,
                  ids_sm, b0, b1, s0, s1):
        sid = lax.axis_index("s")
        c = lax.axis_index("c")
        pltpu.sync_copy(ids_hbm.at[c, sid], ids_sh.at[sid])
        pltpu.sync_copy(ids_sh.at[sid], ids_sm)

        @pl.when(sid == 0)
        def _():
            pltpu.sync_copy(zrows_hbm, tab_sh.at[pl.ds(n_pair, n_sent)])

        bufs = (b0, b1)
        sems = (s0, s1)
        for p in range(nphase):
            pltpu.sync_copy(
                table_hbm.at[pl.ds((2 * c + p) * n_pair + sid * stage,
                                   stage)],
                tab_sh.at[pl.ds(sid * stage, stage)])
            plsc.subcore_barrier()
            base = (c * nphase + p) * (_NS * per_tile) + sid * per_tile
            unroll = 8
            for k in range(nchunks):
                row0 = p * nchunks + k

                def _fire(i, _, _k=k, _row0=row0):
                    for u in range(unroll):
                        j = i * unroll + u
                        rid = ids_sm[_row0 * _CHUNK + j]
                        pltpu.async_copy(tab_sh.at[pl.ds(rid, 1)],
                                         bufs[_k].at[pl.ds(j, 1)], sems[_k])
                    return _

                lax.fori_loop(0, _CHUNK // unroll, _fire, 0)
            for k in range(nchunks):
                pltpu.make_async_copy(table_hbm.at[pl.ds(0, _CHUNK)],
                                      bufs[k], sems[k]).wait()
                pltpu.sync_copy(bufs[k],
                                out_hbm.at[pl.ds(base + k * _CHUNK, _CHUNK)])
            plsc.subcore_barrier()

    return sc_gather


def kernel(x, src_mask, duration_target, pitch_target, energy_target,
           max_len, params, pitch_bins, energy_bins):
    b, src, d = x.shape
    ptab = params["pitch_emb"]
    etab = params["energy_emb"]
    nbins = ptab.shape[0]

    pt3 = pitch_target[:, :, None]
    et3 = energy_target[:, :, None]
    dur3 = duration_target[:, :, None]
    pad = jnp.full((nbins - pitch_bins.shape[0],), jnp.inf, jnp.float32)
    pbrow = jnp.concatenate([pitch_bins, pad])[None, :]
    ebrow = jnp.concatenate([energy_bins, pad])[None, :]

    def _vp_pack(p):
        wa = jnp.transpose(p["w1"], (2, 1, 0))
        wb = jnp.transpose(p["w2"], (2, 1, 0))
        vecs = jnp.stack([p["b1"], p["g1"], p["be1"], p["b2"], p["g2"],
                          p["be2"], p["lw"]])[:, None, :]
        return wa, wb, vecs, p["lb"].reshape(1, 1)

    packs = [_vp_pack(params[k]) for k in ("duration", "pitch", "energy")]
    wa = jnp.stack([pk[0] for pk in packs])
    wb = jnp.stack([pk[1] for pk in packs])
    vecs = jnp.stack([pk[2] for pk in packs])
    lbs = jnp.stack([pk[3] for pk in packs])

    x2, x3flat, g, mask_i = _run_idx(x, pt3, et3, dur3, pbrow, ebrow,
                                     ptab, etab)

    n_sent = 8
    zrows = jnp.zeros((n_sent, d), x.dtype)
    n_rows = b * _MAXLEN
    g_tiled = (g.reshape(2, 2, _NS, 2, _CHUNK)
                .transpose(0, 2, 1, 3, 4)
                .reshape(2, _NS, 4 * _CHUNK))
    out_flat = _make_sc_gather(n_rows, d, src, n_sent)(x3flat, zrows, g_tiled)
    out = out_flat.reshape(b, _MAXLEN, d)

    ld, pp, ep = _run_preds(x, x2, wa, wb, vecs, lbs)
    log_dur = jnp.where(src_mask, 0.0, ld[:, :, 0])
    pitch_pred = jnp.where(src_mask, 0.0, pp[:, :, 0])
    energy_pred = jnp.where(src_mask, 0.0, ep[:, :, 0])

    total = jnp.sum(duration_target, axis=1)
    mel_len = jnp.minimum(total, max_len)
    mel_mask = mask_i.reshape(b, _MAXLEN).astype(bool)

    return out, pitch_pred, energy_pred, log_dur, mel_len, mel_mask
